# baseline (speedup 1.0000x reference)
import functools

import jax
import jax.numpy as jnp
from jax import lax
from jax.experimental import pallas as pl
from jax.experimental.pallas import tpu as pltpu
from jax.experimental.pallas import tpu_sc as plsc

B = 4096
L = 200
EMB = 16
NC = 2
NS = 16
NW = NC * NS
PB = B // NW
G = 8
CHUNK = G * L
NCHUNK = PB // G


def _sc_kernel(uids_h, mids_h, cats_h, midhis_h, cathis_h,
               uid_tab, mid_tab, cat_tab,
               o_uid, o_item, o_his, o_sum,
               sidx, srows, midx, cidx, mbuf, cbuf, acc):
    wid = lax.axis_index("s") * NC + lax.axis_index("c")
    b0 = wid * PB

    pltpu.sync_copy(uids_h.at[pl.ds(b0, PB)], sidx)
    pltpu.sync_copy(uid_tab.at[sidx], srows)
    pltpu.sync_copy(srows, o_uid.at[pl.ds(b0, PB)])

    pltpu.sync_copy(mids_h.at[pl.ds(b0, PB)], sidx)
    pltpu.sync_copy(mid_tab.at[sidx], srows)
    pltpu.sync_copy(srows, o_item.at[pl.ds(b0, PB), 0])

    pltpu.sync_copy(cats_h.at[pl.ds(b0, PB)], sidx)
    pltpu.sync_copy(cat_tab.at[sidx], srows)
    pltpu.sync_copy(srows, o_item.at[pl.ds(b0, PB), 1])

    @pl.loop(0, NCHUNK)
    def _chunk(c):
        r0 = (b0 + c * G) * L
        pltpu.sync_copy(midhis_h.at[pl.ds(r0, CHUNK)], midx)
        pltpu.sync_copy(cathis_h.at[pl.ds(r0, CHUNK)], cidx)
        pltpu.sync_copy(mid_tab.at[midx], mbuf)
        pltpu.sync_copy(cat_tab.at[cidx], cbuf)
        pltpu.sync_copy(mbuf, o_his.at[pl.ds(r0, CHUNK), 0])
        pltpu.sync_copy(cbuf, o_his.at[pl.ds(r0, CHUNK), 1])

        @pl.loop(0, G)
        def _pool(g):
            def body(l, carry):
                am, ac = carry
                r = g * L + l
                return am + mbuf[r, :], ac + cbuf[r, :]

            z = jnp.zeros((EMB,), jnp.float32)
            am, ac = lax.fori_loop(0, L, body, (z, z))
            acc[c * G + g, 0, :] = am
            acc[c * G + g, 1, :] = ac

    pltpu.sync_copy(acc, o_sum.at[pl.ds(b0, PB)])


@jax.jit
def _run(uids, mids, cats, mid_his_flat, cat_his_flat,
         uid_table, mid_table, cat_table):
    mesh = plsc.VectorSubcoreMesh(core_axis_name="c", subcore_axis_name="s")
    f = pl.kernel(
        _sc_kernel,
        mesh=mesh,
        compiler_params=pltpu.CompilerParams(use_tc_tiling_on_sc=False),
        out_type=(
            jax.ShapeDtypeStruct((B, EMB), jnp.float32),
            jax.ShapeDtypeStruct((B, 2, EMB), jnp.float32),
            jax.ShapeDtypeStruct((B * L, 2, EMB), jnp.float32),
            jax.ShapeDtypeStruct((B, 2, EMB), jnp.float32),
        ),
        scratch_types=[
            pltpu.VMEM((PB,), jnp.int32),
            pltpu.VMEM((PB, EMB), jnp.float32),
            pltpu.VMEM((CHUNK,), jnp.int32),
            pltpu.VMEM((CHUNK,), jnp.int32),
            pltpu.VMEM((CHUNK, EMB), jnp.float32),
            pltpu.VMEM((CHUNK, EMB), jnp.float32),
            pltpu.VMEM((PB, 2, EMB), jnp.float32),
        ],
    )
    return f(uids, mids, cats, mid_his_flat, cat_his_flat,
             uid_table, mid_table, cat_table)


def kernel(uids, mids, cats, mid_his, cat_his, mask,
           uid_table, mid_table, cat_table):
    o_uid, o_item, o_his, o_sum = _run(
        uids, mids, cats,
        mid_his.reshape(B * L), cat_his.reshape(B * L),
        uid_table, mid_table, cat_table)
    return (o_uid,
            o_item.reshape(B, 2 * EMB),
            o_his.reshape(B, L, 2 * EMB),
            o_sum.reshape(B, 2 * EMB),
            mask)


# speedup vs baseline: 2.5733x; 2.5733x over previous
import functools

import jax
import jax.numpy as jnp
from jax import lax
from jax.experimental import pallas as pl
from jax.experimental.pallas import tpu as pltpu
from jax.experimental.pallas import tpu_sc as plsc

B = 4096
L = 200
EMB = 16
NC = 2
NS = 16
NW = NC * NS
PB = B // NW
G = 8
CHUNK = G * L
NCHUNK = PB // G


def _sc_kernel(uids_h, mids_h, cats_h, midhis_h, cathis_h,
               uid_tab, mid_tab, cat_tab,
               o_uid, o_item, o_his, o_sum,
               sidx, srows, midx, cidx, mbuf, cbuf, acc):
    wid = lax.axis_index("s") * NC + lax.axis_index("c")
    b0 = wid * PB

    pltpu.sync_copy(uids_h.at[pl.ds(b0, PB)], sidx)
    pltpu.sync_copy(uid_tab.at[sidx], srows)
    pltpu.sync_copy(srows, o_uid.at[pl.ds(b0, PB)])

    pltpu.sync_copy(mids_h.at[pl.ds(b0, PB)], sidx)
    pltpu.sync_copy(mid_tab.at[sidx], srows)
    pltpu.sync_copy(srows, o_item.at[pl.ds(b0, PB), pl.ds(0, EMB)])

    pltpu.sync_copy(cats_h.at[pl.ds(b0, PB)], sidx)
    pltpu.sync_copy(cat_tab.at[sidx], srows)
    pltpu.sync_copy(srows, o_item.at[pl.ds(b0, PB), pl.ds(EMB, EMB)])

    @pl.loop(0, NCHUNK)
    def _chunk(c):
        r0 = (b0 + c * G) * L
        pltpu.sync_copy(midhis_h.at[pl.ds(r0, CHUNK)], midx)
        pltpu.sync_copy(cathis_h.at[pl.ds(r0, CHUNK)], cidx)
        pltpu.sync_copy(mid_tab.at[midx], mbuf)
        pltpu.sync_copy(cat_tab.at[cidx], cbuf)
        pltpu.sync_copy(mbuf, o_his.at[pl.ds(r0, CHUNK), pl.ds(0, EMB)])
        pltpu.sync_copy(cbuf, o_his.at[pl.ds(r0, CHUNK), pl.ds(EMB, EMB)])

        @pl.loop(0, G)
        def _pool(g):
            def body(l, carry):
                am, ac = carry
                r = g * L + l
                return am + mbuf[r, :], ac + cbuf[r, :]

            z = jnp.zeros((EMB,), jnp.float32)
            am, ac = lax.fori_loop(0, L, body, (z, z))
            acc[c * G + g, pl.ds(0, EMB)] = am
            acc[c * G + g, pl.ds(EMB, EMB)] = ac

    pltpu.sync_copy(acc, o_sum.at[pl.ds(b0, PB)])


@jax.jit
def _run(uids, mids, cats, mid_his_flat, cat_his_flat,
         uid_table, mid_table, cat_table):
    mesh = plsc.VectorSubcoreMesh(core_axis_name="c", subcore_axis_name="s")
    f = pl.kernel(
        _sc_kernel,
        mesh=mesh,
        compiler_params=pltpu.CompilerParams(use_tc_tiling_on_sc=False),
        out_type=(
            jax.ShapeDtypeStruct((B, EMB), jnp.float32),
            jax.ShapeDtypeStruct((B, 2 * EMB), jnp.float32),
            jax.ShapeDtypeStruct((B * L, 2 * EMB), jnp.float32),
            jax.ShapeDtypeStruct((B, 2 * EMB), jnp.float32),
        ),
        scratch_types=[
            pltpu.VMEM((PB,), jnp.int32),
            pltpu.VMEM((PB, EMB), jnp.float32),
            pltpu.VMEM((CHUNK,), jnp.int32),
            pltpu.VMEM((CHUNK,), jnp.int32),
            pltpu.VMEM((CHUNK, EMB), jnp.float32),
            pltpu.VMEM((CHUNK, EMB), jnp.float32),
            pltpu.VMEM((PB, 2 * EMB), jnp.float32),
        ],
    )
    return f(uids, mids, cats, mid_his_flat, cat_his_flat,
             uid_table, mid_table, cat_table)


def kernel(uids, mids, cats, mid_his, cat_his, mask,
           uid_table, mid_table, cat_table):
    o_uid, o_item, o_his, o_sum = _run(
        uids, mids, cats,
        mid_his.reshape(B * L), cat_his.reshape(B * L),
        uid_table, mid_table, cat_table)
    return (o_uid,
            o_item,
            o_his.reshape(B, L, 2 * EMB),
            o_sum,
            mask)


# speedup vs baseline: 3.3488x; 1.3014x over previous
import functools

import jax
import jax.numpy as jnp
from jax import lax
from jax.experimental import pallas as pl
from jax.experimental.pallas import tpu as pltpu
from jax.experimental.pallas import tpu_sc as plsc

B = 4096
L = 200
EMB = 16
NC = 2
NS = 16
NW = NC * NS
PB = B // NW
G = 8
CHUNK = G * L
NCHUNK = PB // G


def _sc_kernel(mids_h, cats_h, midhis_h, cathis_h,
               mid_tab, cat_tab,
               o_item, o_his, o_sum,
               sidx, srows, midx, cidx, mbuf, cbuf, acc):
    wid = lax.axis_index("s") * NC + lax.axis_index("c")
    b0 = wid * PB

    pltpu.sync_copy(mids_h.at[pl.ds(b0, PB)], sidx)
    pltpu.sync_copy(mid_tab.at[sidx], srows)
    pltpu.sync_copy(srows, o_item.at[pl.ds(b0, PB), pl.ds(0, EMB)])

    pltpu.sync_copy(cats_h.at[pl.ds(b0, PB)], sidx)
    pltpu.sync_copy(cat_tab.at[sidx], srows)
    pltpu.sync_copy(srows, o_item.at[pl.ds(b0, PB), pl.ds(EMB, EMB)])

    @pl.loop(0, NCHUNK)
    def _chunk(c):
        r0 = (b0 + c * G) * L
        pltpu.sync_copy(midhis_h.at[pl.ds(r0, CHUNK)], midx)
        pltpu.sync_copy(cathis_h.at[pl.ds(r0, CHUNK)], cidx)
        pltpu.sync_copy(mid_tab.at[midx], mbuf)
        pltpu.sync_copy(cat_tab.at[cidx], cbuf)
        pltpu.sync_copy(mbuf, o_his.at[pl.ds(r0, CHUNK), pl.ds(0, EMB)])
        pltpu.sync_copy(cbuf, o_his.at[pl.ds(r0, CHUNK), pl.ds(EMB, EMB)])

        @pl.loop(0, G)
        def _pool(g):
            def body(l, carry):
                am, ac = carry
                r = g * L + l
                return am + mbuf[r, :], ac + cbuf[r, :]

            z = jnp.zeros((EMB,), jnp.float32)
            am, ac = lax.fori_loop(0, L, body, (z, z))
            acc[c * G + g, pl.ds(0, EMB)] = am
            acc[c * G + g, pl.ds(EMB, EMB)] = ac

    pltpu.sync_copy(acc, o_sum.at[pl.ds(b0, PB)])


@jax.jit
def _run(mids, cats, mid_his_flat, cat_his_flat, mid_table, cat_table):
    mesh = plsc.VectorSubcoreMesh(core_axis_name="c", subcore_axis_name="s")
    f = pl.kernel(
        _sc_kernel,
        mesh=mesh,
        compiler_params=pltpu.CompilerParams(use_tc_tiling_on_sc=False),
        out_type=(
            jax.ShapeDtypeStruct((B, 2 * EMB), jnp.float32),
            jax.ShapeDtypeStruct((B * L, 2 * EMB), jnp.float32),
            jax.ShapeDtypeStruct((B, 2 * EMB), jnp.float32),
        ),
        scratch_types=[
            pltpu.VMEM((PB,), jnp.int32),
            pltpu.VMEM((PB, EMB), jnp.float32),
            pltpu.VMEM((CHUNK,), jnp.int32),
            pltpu.VMEM((CHUNK,), jnp.int32),
            pltpu.VMEM((CHUNK, EMB), jnp.float32),
            pltpu.VMEM((CHUNK, EMB), jnp.float32),
            pltpu.VMEM((PB, 2 * EMB), jnp.float32),
        ],
    )
    return f(mids, cats, mid_his_flat, cat_his_flat, mid_table, cat_table)


def kernel(uids, mids, cats, mid_his, cat_his, mask,
           uid_table, mid_table, cat_table):
    o_uid = jnp.take(uid_table, uids, axis=0)
    o_item, o_his, o_sum = _run(
        mids, cats,
        mid_his.reshape(B * L), cat_his.reshape(B * L),
        mid_table, cat_table)
    return (o_uid,
            o_item,
            o_his.reshape(B, L, 2 * EMB),
            o_sum,
            mask)


# speedup vs baseline: 3.4482x; 1.0297x over previous
import functools

import jax
import jax.numpy as jnp
from jax import lax
from jax.experimental import pallas as pl
from jax.experimental.pallas import tpu as pltpu
from jax.experimental.pallas import tpu_sc as plsc

B = 4096
L = 200
EMB = 16
NV = 1000000
NC = 2
NS = 16
NW = NC * NS
PB = B // NW
G = 8
CHUNK = G * L
NCHUNK = PB // G

BLK = 128
NBLK_FULL = NV // BLK
K1_BASE = NBLK_FULL // NW
K1_EXTRA = NBLK_FULL % NW
TAIL = NV - NBLK_FULL * BLK


def _linearize_kernel(tbl_t, tail_t, out):
    wid = lax.axis_index("s") * NC + lax.axis_index("c")
    iota = lax.iota(jnp.int32, 16)

    def do_block(src, src_j0, out_j0, ncol, vbuf, tbuf):
        src_j0 = pl.multiple_of(src_j0, BLK)
        pltpu.sync_copy(src.at[:, pl.ds(src_j0, BLK)], vbuf)
        for j in range(ncol):
            v = plsc.load_gather(vbuf, [iota, jnp.full((16,), j, jnp.int32)])
            tbuf[j // 8, pl.ds((j % 8) * EMB, EMB)] = v
        nrow = ncol * EMB // BLK
        out_r0 = pl.multiple_of(out_j0 * EMB // BLK, EMB)
        pltpu.sync_copy(tbuf.at[pl.ds(0, nrow), :],
                        out.at[pl.ds(out_r0, nrow), :])

    def body(vbuf, tbuf):
        nblk = K1_BASE + jnp.where(wid < K1_EXTRA, 1, 0)

        @pl.loop(0, K1_BASE + 1)
        def _blk(i):
            @pl.when(i < nblk)
            def _():
                j0 = (i * NW + wid) * BLK
                do_block(tbl_t, j0, j0, BLK, vbuf, tbuf)

        @pl.when(wid == K1_EXTRA)
        def _tail():
            do_block(tail_t, 0, NBLK_FULL * BLK, TAIL, vbuf, tbuf)

    pl.run_scoped(body,
                  pltpu.VMEM((EMB, BLK), jnp.float32),
                  pltpu.VMEM((EMB, BLK), jnp.float32))


def _main_kernel(mids_h, cats_h, midhis_h, cathis_h,
                 mid_lin, cat_tab,
                 o_item, o_his, o_sum,
                 sidx, srows, midx, cidx, mbuf, cbuf, acc):
    wid = lax.axis_index("s") * NC + lax.axis_index("c")
    b0 = wid * PB
    mid_tab = mid_lin

    pltpu.sync_copy(mids_h.at[pl.ds(b0, PB)], sidx)
    pltpu.sync_copy(mid_tab.at[sidx], srows)
    pltpu.sync_copy(srows, o_item.at[pl.ds(b0, PB), pl.ds(0, EMB)])

    pltpu.sync_copy(cats_h.at[pl.ds(b0, PB)], sidx)
    pltpu.sync_copy(cat_tab.at[sidx], srows)
    pltpu.sync_copy(srows, o_item.at[pl.ds(b0, PB), pl.ds(EMB, EMB)])

    @pl.loop(0, NCHUNK)
    def _chunk(c):
        r0 = (b0 + c * G) * L
        pltpu.sync_copy(midhis_h.at[pl.ds(r0, CHUNK)], midx)
        pltpu.sync_copy(cathis_h.at[pl.ds(r0, CHUNK)], cidx)
        pltpu.sync_copy(mid_tab.at[midx], mbuf)
        pltpu.sync_copy(cat_tab.at[cidx], cbuf)
        pltpu.sync_copy(mbuf, o_his.at[pl.ds(r0, CHUNK), pl.ds(0, EMB)])
        pltpu.sync_copy(cbuf, o_his.at[pl.ds(r0, CHUNK), pl.ds(EMB, EMB)])

        @pl.loop(0, G)
        def _pool(g):
            def body(l, carry):
                am, ac = carry
                r = g * L + l
                return am + mbuf[r, :], ac + cbuf[r, :]

            z = jnp.zeros((EMB,), jnp.float32)
            am, ac = lax.fori_loop(0, L, body, (z, z))
            acc[c * G + g, pl.ds(0, EMB)] = am
            acc[c * G + g, pl.ds(EMB, EMB)] = ac

    pltpu.sync_copy(acc, o_sum.at[pl.ds(b0, PB)])


@jax.jit
def _run(mids, cats, mid_his_flat, cat_his_flat, mid_table, cat_table):
    mesh = plsc.VectorSubcoreMesh(core_axis_name="c", subcore_axis_name="s")
    linearize = pl.kernel(
        _linearize_kernel,
        mesh=mesh,
        compiler_params=pltpu.CompilerParams(use_tc_tiling_on_sc=True,
                                             needs_layout_passes=False),
        out_type=jax.ShapeDtypeStruct((NV * EMB // BLK, BLK), jnp.float32),
    )
    tbl_t = mid_table.T
    tail_t = jnp.pad(tbl_t[:, NBLK_FULL * BLK:], ((0, 0), (0, BLK - TAIL)))
    mid_lin = linearize(tbl_t, tail_t).reshape(NV, EMB)
    main = pl.kernel(
        _main_kernel,
        mesh=mesh,
        compiler_params=pltpu.CompilerParams(use_tc_tiling_on_sc=False),
        out_type=(
            jax.ShapeDtypeStruct((B, 2 * EMB), jnp.float32),
            jax.ShapeDtypeStruct((B * L, 2 * EMB), jnp.float32),
            jax.ShapeDtypeStruct((B, 2 * EMB), jnp.float32),
        ),
        scratch_types=[
            pltpu.VMEM((PB,), jnp.int32),
            pltpu.VMEM((PB, EMB), jnp.float32),
            pltpu.VMEM((CHUNK,), jnp.int32),
            pltpu.VMEM((CHUNK,), jnp.int32),
            pltpu.VMEM((CHUNK, EMB), jnp.float32),
            pltpu.VMEM((CHUNK, EMB), jnp.float32),
            pltpu.VMEM((PB, 2 * EMB), jnp.float32),
        ],
    )
    return main(mids, cats, mid_his_flat, cat_his_flat, mid_lin, cat_table)


def kernel(uids, mids, cats, mid_his, cat_his, mask,
           uid_table, mid_table, cat_table):
    o_uid = jnp.take(uid_table, uids, axis=0)
    o_item, o_his, o_sum = _run(
        mids, cats,
        mid_his.reshape(B * L), cat_his.reshape(B * L),
        mid_table, cat_table)
    return (o_uid,
            o_item,
            o_his.reshape(B, L, 2 * EMB),
            o_sum,
            mask)


# speedup vs baseline: 3.8742x; 1.1235x over previous
import functools

import jax
import jax.numpy as jnp
from jax import lax
from jax.experimental import pallas as pl
from jax.experimental.pallas import tpu as pltpu
from jax.experimental.pallas import tpu_sc as plsc

B = 4096
L = 200
EMB = 16
NV = 1000000
NC = 2
NS = 16
NW = NC * NS
PB = B // NW
G = 8
CHUNK = G * L
NCHUNK = PB // G

BLK = 512
TROW = BLK * EMB // 128
NBLK_FULL = NV // BLK
K1_BASE = NBLK_FULL // NW
K1_EXTRA = NBLK_FULL % NW
TAIL = NV - NBLK_FULL * BLK
NGRP = (K1_BASE - 1) // 4


def _transpose_block(vbuf, tbuf, iota, ncol):
    zeros = jnp.zeros((16,), jnp.int32)

    @pl.loop(0, ncol // 8)
    def _oct(o):
        base = o * 8
        for k in range(8):
            v = plsc.load_gather(vbuf, [iota, zeros + (base + k)])
            tbuf[o, pl.ds(k * EMB, EMB)] = v


def _linearize_kernel(tbl_t, tail_t, out, vb0, vb1, vb2, vb3,
                      tb0, tb1, tb2, tb3, rsem, wsem):
    wid = lax.axis_index("s") * NC + lax.axis_index("c")
    iota = lax.iota(jnp.int32, 16)
    vbs = (vb0, vb1, vb2, vb3)
    tbs = (tb0, tb1, tb2, tb3)

    def blk_id(i):
        return i * NW + wid

    def read(i, vb):
        j0 = pl.multiple_of(blk_id(i) * BLK, BLK)
        return pltpu.make_async_copy(tbl_t.at[:, pl.ds(j0, BLK)], vb, rsem)

    def write(i, tb):
        r0 = pl.multiple_of(blk_id(i) * TROW, TROW)
        return pltpu.make_async_copy(tb, out.at[pl.ds(r0, TROW), :], wsem)

    for q in range(4):
        read(q, vbs[q]).start()

    @pl.loop(0, NGRP)
    def _grp(g):
        i0 = g * 4
        for q in range(4):
            read(i0 + q, vbs[q]).wait()

            @pl.when(g > 0)
            def _drain():
                write(i0 + q - 4, tbs[q]).wait()

            _transpose_block(vbs[q], tbs[q], iota, BLK)
            write(i0 + q, tbs[q]).start()

            @pl.when(g < NGRP - 1)
            def _next():
                read(i0 + 4 + q, vbs[q]).start()

    for q in range(4):
        write((NGRP - 1) * 4 + q, tbs[q]).wait()

    def solo(i):
        j0 = pl.multiple_of(blk_id(i) * BLK, BLK)
        pltpu.sync_copy(tbl_t.at[:, pl.ds(j0, BLK)], vb0)
        _transpose_block(vb0, tb0, iota, BLK)
        r0 = pl.multiple_of(blk_id(i) * TROW, TROW)
        pltpu.sync_copy(tb0, out.at[pl.ds(r0, TROW), :])

    solo(K1_BASE - 1)

    @pl.when(wid < K1_EXTRA)
    def _extra():
        solo(K1_BASE)

    @pl.when(wid == K1_EXTRA)
    def _tail():
        pltpu.sync_copy(tail_t.at[:, pl.ds(0, 128)], vb0.at[:, pl.ds(0, 128)])
        _transpose_block(vb0, tb0, iota, TAIL)
        nrow = TAIL * EMB // 128
        pltpu.sync_copy(tb0.at[pl.ds(0, nrow), :],
                        out.at[pl.ds(NBLK_FULL * TROW, nrow), :])


def _main_kernel(mids_h, cats_h, midhis_h, cathis_h,
                 mid_tab, cat_tab,
                 o_item, o_his, o_sum,
                 sidx, srows,
                 midxA, cidxA, mbufA, cbufA, gsemA, wsemA,
                 midxB, cidxB, mbufB, cbufB, gsemB, wsemB,
                 acc):
    wid = lax.axis_index("s") * NC + lax.axis_index("c")
    b0 = wid * PB

    pltpu.sync_copy(mids_h.at[pl.ds(b0, PB)], sidx)
    pltpu.sync_copy(mid_tab.at[sidx], srows)
    pltpu.sync_copy(srows, o_item.at[pl.ds(b0, PB), pl.ds(0, EMB)])

    pltpu.sync_copy(cats_h.at[pl.ds(b0, PB)], sidx)
    pltpu.sync_copy(cat_tab.at[sidx], srows)
    pltpu.sync_copy(srows, o_item.at[pl.ds(b0, PB), pl.ds(EMB, EMB)])

    bufsA = (midxA, cidxA, mbufA, cbufA, gsemA, wsemA)
    bufsB = (midxB, cidxB, mbufB, cbufB, gsemB, wsemB)

    def r0_of(c):
        return (b0 + c * G) * L

    def load_idx(c, bufs):
        midx, cidx = bufs[0], bufs[1]
        pltpu.sync_copy(midhis_h.at[pl.ds(r0_of(c), CHUNK)], midx)
        pltpu.sync_copy(cathis_h.at[pl.ds(r0_of(c), CHUNK)], cidx)

    def gathers(c, bufs):
        midx, cidx, mbuf, cbuf, gsem, _ = bufs
        return (pltpu.make_async_copy(mid_tab.at[midx], mbuf, gsem),
                pltpu.make_async_copy(cat_tab.at[cidx], cbuf, gsem))

    def writes(c, bufs):
        _, _, mbuf, cbuf, _, wsem = bufs
        r0 = r0_of(c)
        return (pltpu.make_async_copy(
                    mbuf, o_his.at[pl.ds(r0, CHUNK), pl.ds(0, EMB)], wsem),
                pltpu.make_async_copy(
                    cbuf, o_his.at[pl.ds(r0, CHUNK), pl.ds(EMB, EMB)], wsem))

    def process(c, bufs):
        mbuf, cbuf = bufs[2], bufs[3]
        for cp in gathers(c, bufs):
            cp.wait()
        for cp in writes(c, bufs):
            cp.start()

        @pl.loop(0, G)
        def _pool(g):
            def body(l, carry):
                am, ac = carry
                r = g * L + l
                return am + mbuf[r, :], ac + cbuf[r, :]

            z = jnp.zeros((EMB,), jnp.float32)
            am, ac = lax.fori_loop(0, L, body, (z, z))
            acc[c * G + g, pl.ds(0, EMB)] = am
            acc[c * G + g, pl.ds(EMB, EMB)] = ac

    load_idx(0, bufsA)
    for cp in gathers(0, bufsA):
        cp.start()

    @pl.loop(0, NCHUNK // 2)
    def _pair(p):
        c = p * 2
        load_idx(c + 1, bufsB)

        @pl.when(p > 0)
        def _drainB():
            for cp in writes(c - 1, bufsB):
                cp.wait()

        for cp in gathers(c + 1, bufsB):
            cp.start()

        process(c, bufsA)

        @pl.when(p < NCHUNK // 2 - 1)
        def _nextA():
            load_idx(c + 2, bufsA)
            for cp in writes(c, bufsA):
                cp.wait()
            for cp in gathers(c + 2, bufsA):
                cp.start()

        process(c + 1, bufsB)

    for cp in writes(NCHUNK - 2, bufsA):
        cp.wait()
    for cp in writes(NCHUNK - 1, bufsB):
        cp.wait()
    pltpu.sync_copy(acc, o_sum.at[pl.ds(b0, PB)])


@jax.jit
def _run(mids, cats, mid_his_flat, cat_his_flat, mid_table, cat_table):
    mesh = plsc.VectorSubcoreMesh(core_axis_name="c", subcore_axis_name="s")
    linearize = pl.kernel(
        _linearize_kernel,
        mesh=mesh,
        compiler_params=pltpu.CompilerParams(use_tc_tiling_on_sc=True,
                                             needs_layout_passes=False),
        out_type=jax.ShapeDtypeStruct((NV * EMB // 128, 128), jnp.float32),
        scratch_types=([pltpu.VMEM((EMB, BLK), jnp.float32)] * 4
                       + [pltpu.VMEM((TROW, 128), jnp.float32)] * 4
                       + [pltpu.SemaphoreType.DMA] * 2),
    )
    tbl_t = mid_table.T
    tail_t = jnp.pad(tbl_t[:, NBLK_FULL * BLK:], ((0, 0), (0, 128 - TAIL)))
    mid_lin = linearize(tbl_t, tail_t).reshape(NV, EMB)
    main = pl.kernel(
        _main_kernel,
        mesh=mesh,
        compiler_params=pltpu.CompilerParams(use_tc_tiling_on_sc=False),
        out_type=(
            jax.ShapeDtypeStruct((B, 2 * EMB), jnp.float32),
            jax.ShapeDtypeStruct((B * L, 2 * EMB), jnp.float32),
            jax.ShapeDtypeStruct((B, 2 * EMB), jnp.float32),
        ),
        scratch_types=[
            pltpu.VMEM((PB,), jnp.int32),
            pltpu.VMEM((PB, EMB), jnp.float32),
            pltpu.VMEM((CHUNK,), jnp.int32),
            pltpu.VMEM((CHUNK,), jnp.int32),
            pltpu.VMEM((CHUNK, EMB), jnp.float32),
            pltpu.VMEM((CHUNK, EMB), jnp.float32),
            pltpu.SemaphoreType.DMA,
            pltpu.SemaphoreType.DMA,
            pltpu.VMEM((CHUNK,), jnp.int32),
            pltpu.VMEM((CHUNK,), jnp.int32),
            pltpu.VMEM((CHUNK, EMB), jnp.float32),
            pltpu.VMEM((CHUNK, EMB), jnp.float32),
            pltpu.SemaphoreType.DMA,
            pltpu.SemaphoreType.DMA,
            pltpu.VMEM((PB, 2 * EMB), jnp.float32),
        ],
    )
    return main(mids, cats, mid_his_flat, cat_his_flat, mid_lin, cat_table)


def kernel(uids, mids, cats, mid_his, cat_his, mask,
           uid_table, mid_table, cat_table):
    o_uid = jnp.take(uid_table, uids, axis=0)
    o_item, o_his, o_sum = _run(
        mids, cats,
        mid_his.reshape(B * L), cat_his.reshape(B * L),
        mid_table, cat_table)
    return (o_uid,
            o_item,
            o_his.reshape(B, L, 2 * EMB),
            o_sum,
            mask)


# speedup vs baseline: 3.8824x; 1.0021x over previous
import functools

import jax
import jax.numpy as jnp
from jax import lax
from jax.experimental import pallas as pl
from jax.experimental.pallas import tpu as pltpu
from jax.experimental.pallas import tpu_sc as plsc

B = 4096
L = 200
EMB = 16
NV = 1000000
NC = 2
NS = 16
NW = NC * NS
PB = B // NW
G = 8
CHUNK = G * L
NCHUNK = PB // G

BLK = 512
TROW = BLK * EMB // 128
NBLK_FULL = NV // BLK
K1_BASE = NBLK_FULL // NW
K1_EXTRA = NBLK_FULL % NW
TAIL = NV - NBLK_FULL * BLK
NGRP = (K1_BASE - 1) // 4


def _transpose_block(vbuf, tbuf, iota, ncol):
    zeros = jnp.zeros((16,), jnp.int32)

    @pl.loop(0, ncol // 8)
    def _oct(o):
        base = o * 8
        for k in range(8):
            v = plsc.load_gather(vbuf, [iota, zeros + (base + k)])
            tbuf[o, pl.ds(k * EMB, EMB)] = v


def _linearize_kernel(tbl_t, tail_t, out, vb0, vb1, vb2, vb3,
                      tb0, tb1, tb2, tb3, rsem, wsem):
    wid = lax.axis_index("s") * NC + lax.axis_index("c")
    iota = lax.iota(jnp.int32, 16)
    vbs = (vb0, vb1, vb2, vb3)
    tbs = (tb0, tb1, tb2, tb3)

    def blk_id(i):
        return i * NW + wid

    def read(i, vb):
        j0 = pl.multiple_of(blk_id(i) * BLK, BLK)
        return pltpu.make_async_copy(tbl_t.at[:, pl.ds(j0, BLK)],
                                     vb.at[:, pl.ds(0, BLK)], rsem)

    def write(i, tb):
        r0 = pl.multiple_of(blk_id(i) * TROW, TROW)
        return pltpu.make_async_copy(tb, out.at[pl.ds(r0, TROW), :], wsem)

    for q in range(4):
        read(q, vbs[q]).start()

    @pl.loop(0, NGRP)
    def _grp(g):
        i0 = g * 4
        for q in range(4):
            read(i0 + q, vbs[q]).wait()

            @pl.when(g > 0)
            def _drain():
                write(i0 + q - 4, tbs[q]).wait()

            _transpose_block(vbs[q], tbs[q], iota, BLK)
            write(i0 + q, tbs[q]).start()

            @pl.when(g < NGRP - 1)
            def _next():
                read(i0 + 4 + q, vbs[q]).start()

    for q in range(4):
        write((NGRP - 1) * 4 + q, tbs[q]).wait()

    def solo(i):
        j0 = pl.multiple_of(blk_id(i) * BLK, BLK)
        pltpu.sync_copy(tbl_t.at[:, pl.ds(j0, BLK)], vb0.at[:, pl.ds(0, BLK)])
        _transpose_block(vb0, tb0, iota, BLK)
        r0 = pl.multiple_of(blk_id(i) * TROW, TROW)
        pltpu.sync_copy(tb0, out.at[pl.ds(r0, TROW), :])

    solo(K1_BASE - 1)

    @pl.when(wid < K1_EXTRA)
    def _extra():
        solo(K1_BASE)

    @pl.when(wid == K1_EXTRA)
    def _tail():
        pltpu.sync_copy(tail_t.at[:, pl.ds(0, 128)], vb0.at[:, pl.ds(0, 128)])
        _transpose_block(vb0, tb0, iota, TAIL)
        nrow = TAIL * EMB // 128
        pltpu.sync_copy(tb0.at[pl.ds(0, nrow), :],
                        out.at[pl.ds(NBLK_FULL * TROW, nrow), :])


def _main_kernel(mids_h, cats_h, midhis_h, cathis_h,
                 mid_tab, cat_tab,
                 o_item, o_his, o_sum,
                 sidx, srows,
                 midxA, cidxA, mbufA, cbufA, gsemA, wsemA,
                 midxB, cidxB, mbufB, cbufB, gsemB, wsemB,
                 acc):
    wid = lax.axis_index("s") * NC + lax.axis_index("c")
    b0 = wid * PB

    pltpu.sync_copy(mids_h.at[pl.ds(b0, PB)], sidx)
    pltpu.sync_copy(mid_tab.at[sidx], srows)
    pltpu.sync_copy(srows, o_item.at[pl.ds(b0, PB), pl.ds(0, EMB)])

    pltpu.sync_copy(cats_h.at[pl.ds(b0, PB)], sidx)
    pltpu.sync_copy(cat_tab.at[sidx], srows)
    pltpu.sync_copy(srows, o_item.at[pl.ds(b0, PB), pl.ds(EMB, EMB)])

    bufsA = (midxA, cidxA, mbufA, cbufA, gsemA, wsemA)
    bufsB = (midxB, cidxB, mbufB, cbufB, gsemB, wsemB)

    def r0_of(c):
        return (b0 + c * G) * L

    def load_idx(c, bufs):
        midx, cidx = bufs[0], bufs[1]
        pltpu.sync_copy(midhis_h.at[pl.ds(r0_of(c), CHUNK)], midx)
        pltpu.sync_copy(cathis_h.at[pl.ds(r0_of(c), CHUNK)], cidx)

    def gathers(c, bufs):
        midx, cidx, mbuf, cbuf, gsem, _ = bufs
        return (pltpu.make_async_copy(mid_tab.at[midx], mbuf, gsem),
                pltpu.make_async_copy(cat_tab.at[cidx], cbuf, gsem))

    def writes(c, bufs):
        _, _, mbuf, cbuf, _, wsem = bufs
        r0 = r0_of(c)
        return (pltpu.make_async_copy(
                    mbuf, o_his.at[pl.ds(r0, CHUNK), pl.ds(0, EMB)], wsem),
                pltpu.make_async_copy(
                    cbuf, o_his.at[pl.ds(r0, CHUNK), pl.ds(EMB, EMB)], wsem))

    def process(c, bufs):
        mbuf, cbuf = bufs[2], bufs[3]
        for cp in gathers(c, bufs):
            cp.wait()
        for cp in writes(c, bufs):
            cp.start()

        @pl.loop(0, G)
        def _pool(g):
            def body(l, carry):
                am, ac = carry
                r = g * L + l
                return am + mbuf[r, :], ac + cbuf[r, :]

            z = jnp.zeros((EMB,), jnp.float32)
            am, ac = lax.fori_loop(0, L, body, (z, z))
            acc[c * G + g, pl.ds(0, EMB)] = am
            acc[c * G + g, pl.ds(EMB, EMB)] = ac

    load_idx(0, bufsA)
    for cp in gathers(0, bufsA):
        cp.start()

    @pl.loop(0, NCHUNK // 2)
    def _pair(p):
        c = p * 2
        load_idx(c + 1, bufsB)

        @pl.when(p > 0)
        def _drainB():
            for cp in writes(c - 1, bufsB):
                cp.wait()

        for cp in gathers(c + 1, bufsB):
            cp.start()

        process(c, bufsA)

        @pl.when(p < NCHUNK // 2 - 1)
        def _nextA():
            load_idx(c + 2, bufsA)
            for cp in writes(c, bufsA):
                cp.wait()
            for cp in gathers(c + 2, bufsA):
                cp.start()

        process(c + 1, bufsB)

    for cp in writes(NCHUNK - 2, bufsA):
        cp.wait()
    for cp in writes(NCHUNK - 1, bufsB):
        cp.wait()
    pltpu.sync_copy(acc, o_sum.at[pl.ds(b0, PB)])


@jax.jit
def _run(mids, cats, mid_his_flat, cat_his_flat, mid_table, cat_table):
    mesh = plsc.VectorSubcoreMesh(core_axis_name="c", subcore_axis_name="s")
    linearize = pl.kernel(
        _linearize_kernel,
        mesh=mesh,
        compiler_params=pltpu.CompilerParams(use_tc_tiling_on_sc=True,
                                             needs_layout_passes=False),
        out_type=jax.ShapeDtypeStruct((NV * EMB // 128, 128), jnp.float32),
        scratch_types=([pltpu.VMEM((EMB, BLK + 1), jnp.float32)] * 4
                       + [pltpu.VMEM((TROW, 128), jnp.float32)] * 4
                       + [pltpu.SemaphoreType.DMA] * 2),
    )
    tbl_t = mid_table.T
    tail_t = jnp.pad(tbl_t[:, NBLK_FULL * BLK:], ((0, 0), (0, 128 - TAIL)))
    mid_lin = linearize(tbl_t, tail_t).reshape(NV, EMB)
    main = pl.kernel(
        _main_kernel,
        mesh=mesh,
        compiler_params=pltpu.CompilerParams(use_tc_tiling_on_sc=False),
        out_type=(
            jax.ShapeDtypeStruct((B, 2 * EMB), jnp.float32),
            jax.ShapeDtypeStruct((B * L, 2 * EMB), jnp.float32),
            jax.ShapeDtypeStruct((B, 2 * EMB), jnp.float32),
        ),
        scratch_types=[
            pltpu.VMEM((PB,), jnp.int32),
            pltpu.VMEM((PB, EMB), jnp.float32),
            pltpu.VMEM((CHUNK,), jnp.int32),
            pltpu.VMEM((CHUNK,), jnp.int32),
            pltpu.VMEM((CHUNK, EMB), jnp.float32),
            pltpu.VMEM((CHUNK, EMB), jnp.float32),
            pltpu.SemaphoreType.DMA,
            pltpu.SemaphoreType.DMA,
            pltpu.VMEM((CHUNK,), jnp.int32),
            pltpu.VMEM((CHUNK,), jnp.int32),
            pltpu.VMEM((CHUNK, EMB), jnp.float32),
            pltpu.VMEM((CHUNK, EMB), jnp.float32),
            pltpu.SemaphoreType.DMA,
            pltpu.SemaphoreType.DMA,
            pltpu.VMEM((PB, 2 * EMB), jnp.float32),
        ],
    )
    return main(mids, cats, mid_his_flat, cat_his_flat, mid_lin, cat_table)


def kernel(uids, mids, cats, mid_his, cat_his, mask,
           uid_table, mid_table, cat_table):
    o_uid = jnp.take(uid_table, uids, axis=0)
    o_item, o_his, o_sum = _run(
        mids, cats,
        mid_his.reshape(B * L), cat_his.reshape(B * L),
        mid_table, cat_table)
    return (o_uid,
            o_item,
            o_his.reshape(B, L, 2 * EMB),
            o_sum,
            mask)


# speedup vs baseline: 5.1035x; 1.3145x over previous
import functools

import jax
import jax.numpy as jnp
from jax import lax
from jax.experimental import pallas as pl
from jax.experimental.pallas import tpu as pltpu
from jax.experimental.pallas import tpu_sc as plsc

B = 4096
L = 200
EMB = 16
NV = 1000000
NC = 2
NS = 16
NW = NC * NS
PB = B // NW
G = 8
CHUNK = G * L
NCHUNK = PB // G

BLK = 512
TROW = BLK * EMB // 128
NBLK_FULL = NV // BLK
K1_BASE = NBLK_FULL // NW
K1_EXTRA = NBLK_FULL % NW
TAIL = NV - NBLK_FULL * BLK
NGRP = (K1_BASE - 1) // 4


def _transpose_block(vbuf, tbuf, iota, ncol):
    zeros = jnp.zeros((16,), jnp.int32)

    @pl.loop(0, ncol // 8)
    def _oct(o):
        base = o * 8
        for k in range(8):
            v = plsc.load_gather(vbuf, [iota, zeros + (base + k)])
            tbuf[o, pl.ds(k * EMB, EMB)] = v


def _linearize_kernel(tbl_t, tail_t, out, vb0, vb1, vb2, vb3,
                      tb0, tb1, tb2, tb3, rsem, wsem):
    wid = lax.axis_index("s") * NC + lax.axis_index("c")
    iota = lax.iota(jnp.int32, 16)
    vbs = (vb0, vb1, vb2, vb3)
    tbs = (tb0, tb1, tb2, tb3)

    def blk_id(i):
        return i * NW + wid

    def read(i, vb):
        j0 = pl.multiple_of(blk_id(i) * BLK, BLK)
        return pltpu.make_async_copy(tbl_t.at[:, pl.ds(j0, BLK)],
                                     vb.at[:, pl.ds(0, BLK)], rsem)

    def write(i, tb):
        r0 = pl.multiple_of(blk_id(i) * TROW, TROW)
        return pltpu.make_async_copy(tb, out.at[pl.ds(r0, TROW), :], wsem)

    for q in range(4):
        read(q, vbs[q]).start()

    @pl.loop(0, NGRP)
    def _grp(g):
        i0 = g * 4
        for q in range(4):
            read(i0 + q, vbs[q]).wait()

            @pl.when(g > 0)
            def _drain():
                write(i0 + q - 4, tbs[q]).wait()

            _transpose_block(vbs[q], tbs[q], iota, BLK)
            write(i0 + q, tbs[q]).start()

            @pl.when(g < NGRP - 1)
            def _next():
                read(i0 + 4 + q, vbs[q]).start()

    for q in range(4):
        write((NGRP - 1) * 4 + q, tbs[q]).wait()

    def solo(i):
        j0 = pl.multiple_of(blk_id(i) * BLK, BLK)
        pltpu.sync_copy(tbl_t.at[:, pl.ds(j0, BLK)], vb0.at[:, pl.ds(0, BLK)])
        _transpose_block(vb0, tb0, iota, BLK)
        r0 = pl.multiple_of(blk_id(i) * TROW, TROW)
        pltpu.sync_copy(tb0, out.at[pl.ds(r0, TROW), :])

    solo(K1_BASE - 1)

    @pl.when(wid < K1_EXTRA)
    def _extra():
        solo(K1_BASE)

    @pl.when(wid == K1_EXTRA)
    def _tail():
        pltpu.sync_copy(tail_t.at[:, pl.ds(0, 128)], vb0.at[:, pl.ds(0, 128)])
        _transpose_block(vb0, tb0, iota, TAIL)
        nrow = TAIL * EMB // 128
        pltpu.sync_copy(tb0.at[pl.ds(0, nrow), :],
                        out.at[pl.ds(NBLK_FULL * TROW, nrow), :])


def _main_kernel(mids_h, cats_h, midhis_h, cathis_h,
                 mid_tab, cat_tab,
                 o_item, o_his, o_sum,
                 sidx, srows,
                 midxA, cidxA, mbufA, cbufA, gsemA, wsemA,
                 midxB, cidxB, mbufB, cbufB, gsemB, wsemB,
                 acc):
    wid = lax.axis_index("s") * NC + lax.axis_index("c")
    b0 = wid * PB

    pltpu.sync_copy(mids_h.at[pl.ds(b0, PB)], sidx)
    pltpu.sync_copy(mid_tab.at[sidx], srows)
    pltpu.sync_copy(srows, o_item.at[pl.ds(b0, PB), pl.ds(0, EMB)])

    pltpu.sync_copy(cats_h.at[pl.ds(b0, PB)], sidx)
    pltpu.sync_copy(cat_tab.at[sidx], srows)
    pltpu.sync_copy(srows, o_item.at[pl.ds(b0, PB), pl.ds(EMB, EMB)])

    bufsA = (midxA, cidxA, mbufA, cbufA, gsemA, wsemA)
    bufsB = (midxB, cidxB, mbufB, cbufB, gsemB, wsemB)

    def r0_of(c):
        return (b0 + c * G) * L

    def load_idx(c, bufs):
        midx, cidx = bufs[0], bufs[1]
        pltpu.sync_copy(midhis_h.at[pl.ds(r0_of(c), CHUNK)], midx)
        pltpu.sync_copy(cathis_h.at[pl.ds(r0_of(c), CHUNK)], cidx)

    def gathers(c, bufs):
        midx, cidx, mbuf, cbuf, gsem, _ = bufs
        return (pltpu.make_async_copy(mid_tab.at[midx], mbuf, gsem),
                pltpu.make_async_copy(cat_tab.at[cidx], cbuf, gsem))

    def writes(c, bufs):
        _, _, mbuf, cbuf, _, wsem = bufs
        r0 = r0_of(c)
        return (pltpu.make_async_copy(
                    mbuf, o_his.at[pl.ds(r0, CHUNK), pl.ds(0, EMB)], wsem),
                pltpu.make_async_copy(
                    cbuf, o_his.at[pl.ds(r0, CHUNK), pl.ds(EMB, EMB)], wsem))

    def process(c, bufs):
        mbuf, cbuf = bufs[2], bufs[3]
        for cp in gathers(c, bufs):
            cp.wait()
        for cp in writes(c, bufs):
            cp.start()

        @pl.loop(0, G)
        def _pool(g):
            def body(l, carry):
                am, ac = carry
                r = g * L + l
                return am + mbuf[r, :], ac + cbuf[r, :]

            z = jnp.zeros((EMB,), jnp.float32)
            am, ac = lax.fori_loop(0, L, body, (z, z))
            acc[c * G + g, pl.ds(0, EMB)] = am
            acc[c * G + g, pl.ds(EMB, EMB)] = ac

    load_idx(0, bufsA)
    for cp in gathers(0, bufsA):
        cp.start()

    @pl.loop(0, NCHUNK // 2)
    def _pair(p):
        c = p * 2
        load_idx(c + 1, bufsB)

        @pl.when(p > 0)
        def _drainB():
            for cp in writes(c - 1, bufsB):
                cp.wait()

        for cp in gathers(c + 1, bufsB):
            cp.start()

        process(c, bufsA)

        @pl.when(p < NCHUNK // 2 - 1)
        def _nextA():
            load_idx(c + 2, bufsA)
            for cp in writes(c, bufsA):
                cp.wait()
            for cp in gathers(c + 2, bufsA):
                cp.start()

        process(c + 1, bufsB)

    for cp in writes(NCHUNK - 2, bufsA):
        cp.wait()
    for cp in writes(NCHUNK - 1, bufsB):
        cp.wait()
    pltpu.sync_copy(acc, o_sum.at[pl.ds(b0, PB)])


@jax.jit
def _run(mids, cats, mid_his_flat, cat_his_flat, mid_table, cat_table):
    mesh = plsc.VectorSubcoreMesh(core_axis_name="c", subcore_axis_name="s")
    linearize = pl.kernel(
        _linearize_kernel,
        mesh=mesh,
        compiler_params=pltpu.CompilerParams(use_tc_tiling_on_sc=True,
                                             needs_layout_passes=False),
        out_type=jax.ShapeDtypeStruct((NV * EMB // 128, 128), jnp.float32),
        scratch_types=([pltpu.VMEM((EMB, BLK + 1), jnp.float32)] * 4
                       + [pltpu.VMEM((TROW, 128), jnp.float32)] * 4
                       + [pltpu.SemaphoreType.DMA] * 2),
    )
    tbl_t = mid_table.T
    tail_t = jnp.pad(tbl_t[:, NBLK_FULL * BLK:], ((0, 0), (0, 128 - TAIL)))
    mid_lin = linearize(tbl_t, tail_t).reshape(NV, EMB)
    main = pl.kernel(
        _main_kernel,
        mesh=mesh,
        compiler_params=pltpu.CompilerParams(use_tc_tiling_on_sc=False),
        out_type=(
            jax.ShapeDtypeStruct((B, 2 * EMB), jnp.float32),
            jax.ShapeDtypeStruct((B * L, 128), jnp.float32),
            jax.ShapeDtypeStruct((B, 2 * EMB), jnp.float32),
        ),
        scratch_types=[
            pltpu.VMEM((PB,), jnp.int32),
            pltpu.VMEM((PB, EMB), jnp.float32),
            pltpu.VMEM((CHUNK,), jnp.int32),
            pltpu.VMEM((CHUNK,), jnp.int32),
            pltpu.VMEM((CHUNK, EMB), jnp.float32),
            pltpu.VMEM((CHUNK, EMB), jnp.float32),
            pltpu.SemaphoreType.DMA,
            pltpu.SemaphoreType.DMA,
            pltpu.VMEM((CHUNK,), jnp.int32),
            pltpu.VMEM((CHUNK,), jnp.int32),
            pltpu.VMEM((CHUNK, EMB), jnp.float32),
            pltpu.VMEM((CHUNK, EMB), jnp.float32),
            pltpu.SemaphoreType.DMA,
            pltpu.SemaphoreType.DMA,
            pltpu.VMEM((PB, 2 * EMB), jnp.float32),
        ],
    )
    return main(mids, cats, mid_his_flat, cat_his_flat, mid_lin, cat_table)


def kernel(uids, mids, cats, mid_his, cat_his, mask,
           uid_table, mid_table, cat_table):
    o_uid = jnp.take(uid_table, uids, axis=0)
    o_item, o_his, o_sum = _run(
        mids, cats,
        mid_his.reshape(B * L), cat_his.reshape(B * L),
        mid_table, cat_table)
    return (o_uid,
            o_item,
            o_his.reshape(B, L, 128)[:, :, :2 * EMB],
            o_sum,
            mask)
